# baseline (speedup 1.0000x reference)
import functools

import jax
import jax.numpy as jnp
from jax import lax
from jax.experimental import pallas as pl
from jax.experimental.pallas import tpu as pltpu
from jax.experimental.pallas import tpu_sc as plsc

_VOCAB = 32100
_EMB = 32
_HID = 2048
_B = 4096
_L = 50

_NW = 32
_SEGS = 2 * _B
_SEG_PER_W = _SEGS // _NW
_CH = 2
_ROWS = _CH * _L
_NCHUNK = _SEG_PER_W // _CH
_NBUF = 8


def _make_sc_body(nchunk, seg_per_w):
    def _sc_body(ids_hbm, emb_hbm, out_hbm, idx_v, rows_v, acc_v, sems):
        wid = lax.axis_index("s") * 2 + lax.axis_index("c")
        pltpu.sync_copy(ids_hbm.at[wid], idx_v)
        for b in range(_NBUF):
            pltpu.async_copy(emb_hbm.at[idx_v.at[b]], rows_v.at[b], sems.at[b])

        def group(g, carry):
            c0 = g * _NBUF
            for b in range(_NBUF):
                c = c0 + b
                pltpu.make_async_copy(
                    emb_hbm.at[idx_v.at[c]], rows_v.at[b], sems.at[b]).wait()
                for s in range(_CH):
                    seg = c * _CH + s
                    a0 = rows_v[b, s * _L, 0:16]
                    a1 = rows_v[b, s * _L, 16:32]
                    for l in range(1, _L):
                        a0 = a0 + rows_v[b, s * _L + l, 0:16]
                        a1 = a1 + rows_v[b, s * _L + l, 16:32]
                    acc_v[seg, 0:16] = a0
                    acc_v[seg, 16:32] = a1

                @pl.when(c + _NBUF < nchunk)
                def _():
                    pltpu.async_copy(
                        emb_hbm.at[idx_v.at[c + _NBUF]], rows_v.at[b],
                        sems.at[b])
            return carry

        lax.fori_loop(0, nchunk // _NBUF, group, 0)
        pltpu.sync_copy(acc_v, out_hbm.at[pl.ds(wid * seg_per_w, seg_per_w)])

    return _sc_body


def _sc_pool(ids3d, emb):
    nchunk = ids3d.shape[1]
    seg_per_w = nchunk * _CH
    mesh = plsc.VectorSubcoreMesh(core_axis_name="c", subcore_axis_name="s")
    return pl.kernel(
        _make_sc_body(nchunk, seg_per_w),
        out_type=jax.ShapeDtypeStruct((_NW * seg_per_w, _EMB), jnp.float32),
        mesh=mesh,
        scratch_types=[
            pltpu.VMEM((nchunk, _ROWS), jnp.int32),
            pltpu.VMEM((_NBUF, _ROWS, _EMB), jnp.float32),
            pltpu.VMEM((seg_per_w, _EMB), jnp.float32),
            pltpu.SemaphoreType.DMA((_NBUF,)),
        ],
        compiler_params=pltpu.CompilerParams(use_tc_tiling_on_sc=False),
    )(ids3d, emb)


def _tc_indep_body(do, di, nu, dW, db, nW, nb, out):
    j = pl.program_id(1)
    xd = jnp.where(j == 0, do[...], di[...])
    rd = jnp.maximum(
        jnp.dot(xd, dW[...], preferred_element_type=jnp.float32) + db[...], 0.0)
    rn = jnp.maximum(
        jnp.dot(nu[...], nW[...], preferred_element_type=jnp.float32) + nb[...], 0.0)
    out[...] = jnp.where(j < 2, rd, rn)


def _tc_indep(domain_out, domain_in, numerics, dW, db, nW, nb, block_b=1024):
    nsteps = _B // block_b
    full = lambda shape: pl.BlockSpec(shape, lambda i, j: (0, 0))
    bspec = lambda w: pl.BlockSpec((block_b, w), lambda i, j: (i, 0))
    return pl.pallas_call(
        _tc_indep_body,
        grid=(nsteps, 3),
        in_specs=[
            bspec(64), bspec(64), bspec(8),
            full((64, _HID)), full((1, _HID)),
            full((8, _HID)), full((1, _HID)),
        ],
        out_specs=pl.BlockSpec((block_b, _HID), lambda i, j: (i, j + 2)),
        out_shape=jax.ShapeDtypeStruct((_B, 5 * _HID), jnp.float32),
        compiler_params=pltpu.CompilerParams(
            dimension_semantics=("parallel", "arbitrary"),
        ),
    )(domain_out, domain_in, numerics,
      dW, db.reshape(1, _HID), nW, nb.reshape(1, _HID))


def _tc_anchor_body(prev, pa, aW, ab, out):
    del prev
    scale = jnp.float32(1.0 / _L)
    out[...] = jnp.maximum(
        jnp.dot(pa[...] * scale, aW[...],
                preferred_element_type=jnp.float32) + ab[...], 0.0)


def _tc_anchor(prev, pooled_half, aW, ab, col, block_b=1024):
    nsteps = _B // block_b
    return pl.pallas_call(
        _tc_anchor_body,
        grid=(nsteps,),
        in_specs=[
            pl.BlockSpec(memory_space=pltpu.HBM),
            pl.BlockSpec((block_b, _EMB), lambda i: (i, 0)),
            pl.BlockSpec((_EMB, _HID), lambda i: (0, 0)),
            pl.BlockSpec((1, _HID), lambda i: (0, 0)),
        ],
        out_specs=pl.BlockSpec((block_b, _HID), lambda i: (i, col)),
        out_shape=jax.ShapeDtypeStruct((_B, 5 * _HID), jnp.float32),
        input_output_aliases={0: 0},
        compiler_params=pltpu.CompilerParams(
            dimension_semantics=("parallel",),
        ),
    )(prev, pooled_half, aW, ab.reshape(1, _HID))


def kernel(anchor_out_ids, anchor_in_ids, domain_out, domain_in, numerics,
           emb, aW, ab, dW, db, nW, nb):
    nch = _B // (_NW * _CH)
    ids_lo = anchor_out_ids.reshape(-1).astype(jnp.int32).reshape(
        _NW, nch, _ROWS)
    ids_hi = anchor_in_ids.reshape(-1).astype(jnp.int32).reshape(
        _NW, nch, _ROWS)
    pooled_lo = _sc_pool(ids_lo, emb)
    pooled_hi = _sc_pool(ids_hi, emb)
    partial_out = _tc_indep(domain_out.astype(jnp.float32),
                            domain_in.astype(jnp.float32), numerics,
                            dW, db, nW, nb)
    out1 = _tc_anchor(partial_out, pooled_lo, aW, ab, col=0)
    out2 = _tc_anchor(out1, pooled_hi, aW, ab, col=1)
    return out2.reshape(_B, 5, _HID)


# speedup vs baseline: 1.0142x; 1.0142x over previous
import functools

import jax
import jax.numpy as jnp
from jax import lax
from jax.experimental import pallas as pl
from jax.experimental.pallas import tpu as pltpu
from jax.experimental.pallas import tpu_sc as plsc

_VOCAB = 32100
_EMB = 32
_HID = 2048
_B = 4096
_L = 50

_NW = 32
_SEGS = 2 * _B
_SEG_PER_W = _SEGS // _NW
_CH = 2
_ROWS = _CH * _L
_NCHUNK = _SEG_PER_W // _CH
_NBUF = 8


def _sc_body(ids_hbm, emb_hbm, out_hbm, idx_v, rows_v, acc_v, sems):
    wid = lax.axis_index("s") * 2 + lax.axis_index("c")
    pltpu.sync_copy(ids_hbm.at[wid], idx_v)
    for b in range(_NBUF):
        pltpu.async_copy(emb_hbm.at[idx_v.at[b]], rows_v.at[b], sems.at[b])

    def group(g, carry):
        c0 = g * _NBUF
        for b in range(_NBUF):
            c = c0 + b
            pltpu.make_async_copy(
                emb_hbm.at[idx_v.at[c]], rows_v.at[b], sems.at[b]).wait()
            for s in range(_CH):
                seg = c * _CH + s
                a0 = rows_v[b, s * _L, 0:16]
                a1 = rows_v[b, s * _L, 16:32]
                for l in range(1, _L):
                    a0 = a0 + rows_v[b, s * _L + l, 0:16]
                    a1 = a1 + rows_v[b, s * _L + l, 16:32]
                acc_v[seg, 0:16] = a0
                acc_v[seg, 16:32] = a1

            @pl.when(c + _NBUF < _NCHUNK)
            def _():
                pltpu.async_copy(
                    emb_hbm.at[idx_v.at[c + _NBUF]], rows_v.at[b], sems.at[b])
        return carry

    lax.fori_loop(0, _NCHUNK // _NBUF, group, 0)
    pltpu.sync_copy(acc_v, out_hbm.at[pl.ds(wid * _SEG_PER_W, _SEG_PER_W)])


def _sc_pool(ids3d, emb):
    mesh = plsc.VectorSubcoreMesh(core_axis_name="c", subcore_axis_name="s")
    return pl.kernel(
        _sc_body,
        out_type=jax.ShapeDtypeStruct((_SEGS, _EMB), jnp.float32),
        mesh=mesh,
        scratch_types=[
            pltpu.VMEM((_NCHUNK, _ROWS), jnp.int32),
            pltpu.VMEM((_NBUF, _ROWS, _EMB), jnp.float32),
            pltpu.VMEM((_SEG_PER_W, _EMB), jnp.float32),
            pltpu.SemaphoreType.DMA((_NBUF,)),
        ],
        compiler_params=pltpu.CompilerParams(use_tc_tiling_on_sc=False),
    )(ids3d, emb)


def _tc_indep_body(do, di, nu, dW, db, nW, nb, out):
    j = pl.program_id(1)
    xd = jnp.where(j == 0, do[...], di[...])
    rd = jnp.maximum(
        jnp.dot(xd, dW[...], preferred_element_type=jnp.float32) + db[...], 0.0)
    rn = jnp.maximum(
        jnp.dot(nu[...], nW[...], preferred_element_type=jnp.float32) + nb[...], 0.0)
    out[...] = jnp.where(j < 2, rd, rn)


def _tc_indep(domain_out, domain_in, numerics, dW, db, nW, nb, block_b=1024):
    nsteps = _B // block_b
    full = lambda shape: pl.BlockSpec(shape, lambda i, j: (0, 0))
    bspec = lambda w: pl.BlockSpec((block_b, w), lambda i, j: (i, 0))
    return pl.pallas_call(
        _tc_indep_body,
        grid=(nsteps, 3),
        in_specs=[
            bspec(64), bspec(64), bspec(8),
            full((64, _HID)), full((1, _HID)),
            full((8, _HID)), full((1, _HID)),
        ],
        out_specs=pl.BlockSpec((block_b, _HID), lambda i, j: (i, j + 2)),
        out_shape=jax.ShapeDtypeStruct((_B, 5 * _HID), jnp.float32),
        compiler_params=pltpu.CompilerParams(
            dimension_semantics=("parallel", "arbitrary"),
        ),
    )(domain_out, domain_in, numerics,
      dW, db.reshape(1, _HID), nW, nb.reshape(1, _HID))


def _tc_anchor_body(prev, pa, pi, aW, ab, out):
    del prev
    j = pl.program_id(1)
    scale = jnp.float32(1.0 / _L)
    x = jnp.where(j == 0, pa[...], pi[...]) * scale
    out[...] = jnp.maximum(
        jnp.dot(x, aW[...], preferred_element_type=jnp.float32) + ab[...], 0.0)


def _tc_anchor(prev, pooled, aW, ab, block_b=1024):
    nsteps = _B // block_b
    return pl.pallas_call(
        _tc_anchor_body,
        grid=(nsteps, 2),
        in_specs=[
            pl.BlockSpec(memory_space=pltpu.HBM),
            pl.BlockSpec((block_b, _EMB), lambda i, j: (i, 0)),
            pl.BlockSpec((block_b, _EMB), lambda i, j: (i + nsteps, 0)),
            pl.BlockSpec((_EMB, _HID), lambda i, j: (0, 0)),
            pl.BlockSpec((1, _HID), lambda i, j: (0, 0)),
        ],
        out_specs=pl.BlockSpec((block_b, _HID), lambda i, j: (i, j)),
        out_shape=jax.ShapeDtypeStruct((_B, 5 * _HID), jnp.float32),
        input_output_aliases={0: 0},
        compiler_params=pltpu.CompilerParams(
            dimension_semantics=("parallel", "arbitrary"),
        ),
    )(prev, pooled, pooled, aW, ab.reshape(1, _HID))


def kernel(anchor_out_ids, anchor_in_ids, domain_out, domain_in, numerics,
           emb, aW, ab, dW, db, nW, nb):
    ids3d = jnp.concatenate(
        [anchor_out_ids.reshape(-1), anchor_in_ids.reshape(-1)]
    ).astype(jnp.int32).reshape(_NW, _NCHUNK, _ROWS)
    pooled = _sc_pool(ids3d, emb)
    partial_out = _tc_indep(domain_out.astype(jnp.float32),
                            domain_in.astype(jnp.float32), numerics,
                            dW, db, nW, nb)
    out = _tc_anchor(partial_out, pooled, aW, ab)
    return out.reshape(_B, 5, _HID)
